# Initial kernel scaffold; baseline (speedup 1.0000x reference)
import jax
import jax.numpy as jnp
from jax.experimental import pallas as pl


def kernel(x):
    raise NotImplementedError("write your pallas kernel here")


# speedup vs baseline: 1.2409x; 1.2409x over previous
import functools

import jax
import jax.numpy as jnp
import numpy as np
from jax import lax
from jax.experimental import pallas as pl
from jax.experimental.pallas import tpu as pltpu
from jax.experimental.pallas import tpu_sc as plsc

_H, _W = 32, 32
_B, _D = 64, 768
_T = _H * _W
_ROWS = _B * _T

_NC, _NS = 2, 16
_NW = _NC * _NS
_ROWS_W = _ROWS // _NW
_K = 64
_NCHUNK = _ROWS_W // _K


def _zigzag_order(h, w):
    order = []
    for i in range(h):
        cols = range(w) if i % 2 == 0 else range(w - 1, -1, -1)
        order.extend(i * w + j for j in cols)
    return np.array(order, dtype=np.int32)


_SRC_ROWS = (
    np.arange(_B, dtype=np.int32)[:, None] * _T
    + _zigzag_order(_H, _W)[None, :]
).reshape(-1)

_MESH = plsc.VectorSubcoreMesh(
    core_axis_name="c", subcore_axis_name="s",
    num_cores=_NC, num_subcores=_NS,
)


@functools.partial(
    pl.kernel,
    out_type=jax.ShapeDtypeStruct((_ROWS, _D), jnp.float32),
    mesh=_MESH,
    scratch_types=[
        pltpu.VMEM((_K,), jnp.int32),
        pltpu.VMEM((_K, _D), jnp.float32),
        pltpu.SemaphoreType.DMA,
    ],
)
def _zigzag_sc(x_hbm, idx_hbm, out_hbm, idx_v, buf_v, sem):
    wid = lax.axis_index("s") * _NC + lax.axis_index("c")
    base = wid * _ROWS_W

    def step(i, carry):
        off = base + i * _K
        pltpu.sync_copy(idx_hbm.at[pl.ds(off, _K)], idx_v)
        pltpu.async_copy(x_hbm.at[idx_v], buf_v, sem).wait()
        pltpu.sync_copy(buf_v, out_hbm.at[pl.ds(off, _K)])
        return carry

    lax.fori_loop(0, _NCHUNK, step, 0)


def kernel(x):
    x2 = x.reshape(_ROWS, _D)
    idx = jnp.asarray(_SRC_ROWS)
    out = _zigzag_sc(x2, idx)
    return out.reshape(_B, _T, _D)


# speedup vs baseline: 1.4471x; 1.1661x over previous
import functools

import jax
import jax.numpy as jnp
import numpy as np
from jax import lax
from jax.experimental import pallas as pl
from jax.experimental.pallas import tpu as pltpu
from jax.experimental.pallas import tpu_sc as plsc

_H, _W = 32, 32
_B, _D = 64, 768
_T = _H * _W
_ROWS = _B * _T

_NC, _NS = 2, 16
_NW = _NC * _NS
_ROWS_W = _ROWS // _NW
_K = 64
_NCHUNK = _ROWS_W // _K
_G = _NCHUNK // 2


def _zigzag_order(h, w):
    order = []
    for i in range(h):
        cols = range(w) if i % 2 == 0 else range(w - 1, -1, -1)
        order.extend(i * w + j for j in cols)
    return np.array(order, dtype=np.int32)


_SRC_ROWS = (
    np.arange(_B, dtype=np.int32)[:, None] * _T
    + _zigzag_order(_H, _W)[None, :]
).reshape(_NW, _NCHUNK, _K)

_MESH = plsc.VectorSubcoreMesh(
    core_axis_name="c", subcore_axis_name="s",
    num_cores=_NC, num_subcores=_NS,
)


@functools.partial(
    pl.kernel,
    out_type=jax.ShapeDtypeStruct((_ROWS, _D), jnp.float32),
    mesh=_MESH,
    scratch_types=[
        pltpu.VMEM((_NCHUNK, _K), jnp.int32),
        pltpu.VMEM((_K, _D), jnp.float32),
        pltpu.VMEM((_K, _D), jnp.float32),
        pltpu.SemaphoreType.DMA,
        pltpu.SemaphoreType.DMA,
        pltpu.SemaphoreType.DMA,
        pltpu.SemaphoreType.DMA,
    ],
)
def _zigzag_sc(x_hbm, idx_hbm, out_hbm, idx_v, buf0, buf1,
               sem_in0, sem_in1, sem_out0, sem_out1):
    wid = lax.axis_index("s") * _NC + lax.axis_index("c")
    base = wid * _ROWS_W

    bufs = (buf0, buf1)
    sems_in = (sem_in0, sem_in1)
    sems_out = (sem_out0, sem_out1)

    pltpu.sync_copy(idx_hbm.at[wid], idx_v)

    def start_in(i, b):
        pltpu.make_async_copy(x_hbm.at[idx_v.at[i]], bufs[b], sems_in[b]).start()

    def wait_in(b):
        pltpu.make_async_copy(x_hbm.at[idx_v.at[0]], bufs[b], sems_in[b]).wait()

    def start_out(i, b):
        pltpu.make_async_copy(
            bufs[b], out_hbm.at[pl.ds(base + i * _K, _K)], sems_out[b]).start()

    def wait_out(b):
        pltpu.make_async_copy(
            bufs[b], out_hbm.at[pl.ds(base, _K)], sems_out[b]).wait()

    start_in(0, 0)

    def outer(g, carry):
        i0 = 2 * g
        wait_in(0)

        @pl.when(g > 0)
        def _():
            wait_out(1)

        start_in(i0 + 1, 1)
        start_out(i0, 0)

        wait_in(1)
        wait_out(0)

        @pl.when(g < _G - 1)
        def _():
            start_in(i0 + 2, 0)

        start_out(i0 + 1, 1)
        return carry

    lax.fori_loop(0, _G, outer, 0)
    wait_out(1)


def kernel(x):
    x2 = x.reshape(_ROWS, _D)
    idx = jnp.asarray(_SRC_ROWS)
    out = _zigzag_sc(x2, idx)
    return out.reshape(_B, _T, _D)


# speedup vs baseline: 1.4558x; 1.0060x over previous
import functools

import jax
import jax.numpy as jnp
import numpy as np
from jax import lax
from jax.experimental import pallas as pl
from jax.experimental.pallas import tpu as pltpu
from jax.experimental.pallas import tpu_sc as plsc

_H, _W = 32, 32
_B, _D = 64, 768
_T = _H * _W
_ROWS = _B * _T

_NC, _NS = 2, 16
_NW = _NC * _NS
_ROWS_W = _ROWS // _NW
_K = 32
_NB = 4
_NCHUNK = _ROWS_W // _K
_G = _NCHUNK // _NB


def _zigzag_order(h, w):
    order = []
    for i in range(h):
        cols = range(w) if i % 2 == 0 else range(w - 1, -1, -1)
        order.extend(i * w + j for j in cols)
    return np.array(order, dtype=np.int32)


_SRC_ROWS = (
    np.arange(_B, dtype=np.int32)[:, None] * _T
    + _zigzag_order(_H, _W)[None, :]
).reshape(_NW, _NCHUNK, _K)

_MESH = plsc.VectorSubcoreMesh(
    core_axis_name="c", subcore_axis_name="s",
    num_cores=_NC, num_subcores=_NS,
)


@functools.partial(
    pl.kernel,
    out_type=jax.ShapeDtypeStruct((_ROWS, _D), jnp.float32),
    mesh=_MESH,
    scratch_types=[
        pltpu.VMEM((_NCHUNK, _K), jnp.int32),
    ] + [pltpu.VMEM((_K, _D), jnp.float32) for _ in range(_NB)]
      + [pltpu.SemaphoreType.DMA for _ in range(2 * _NB)],
)
def _zigzag_sc(x_hbm, idx_hbm, out_hbm, idx_v, *rest):
    bufs = rest[:_NB]
    sems_in = rest[_NB:2 * _NB]
    sems_out = rest[2 * _NB:]

    wid = lax.axis_index("s") * _NC + lax.axis_index("c")
    base = wid * _ROWS_W

    pltpu.sync_copy(idx_hbm.at[wid], idx_v)

    def start_in(i, b):
        pltpu.make_async_copy(x_hbm.at[idx_v.at[i]], bufs[b], sems_in[b]).start()

    def wait_in(b):
        pltpu.make_async_copy(x_hbm.at[idx_v.at[0]], bufs[b], sems_in[b]).wait()

    def start_out(i, b):
        pltpu.make_async_copy(
            bufs[b], out_hbm.at[pl.ds(base + i * _K, _K)], sems_out[b]).start()

    def wait_out(b):
        pltpu.make_async_copy(
            bufs[b], out_hbm.at[pl.ds(base, _K)], sems_out[b]).wait()

    for b in range(_NB - 1):
        start_in(b, b)

    def outer(g, carry):
        for b in range(_NB):
            i = g * _NB + b
            wait_in(b)
            start_out(i, b)
            bj = (b + _NB - 1) % _NB
            if b == 0:
                @pl.when(g > 0)
                def _():
                    wait_out(bj)

                start_in(i + _NB - 1, bj)
            else:
                @pl.when(g < _G - 1)
                def _():
                    wait_out(bj)
                    start_in(i + _NB - 1, bj)
        return carry

    lax.fori_loop(0, _G, outer, 0)
    for b in range(_NB):
        wait_out(b)


def kernel(x):
    x2 = x.reshape(_ROWS, _D)
    idx = jnp.asarray(_SRC_ROWS)
    out = _zigzag_sc(x2, idx)
    return out.reshape(_B, _T, _D)
